# baseline (speedup 1.0000x reference)
import functools

import jax
import jax.numpy as jnp
from jax import lax
from jax.experimental import pallas as pl
from jax.experimental.pallas import tpu as pltpu
from jax.experimental.pallas import tpu_sc as plsc

TOPK = 8


def _scores_body(x1_ref, x2_ref, wt_ref, s_out_ref):
    h2 = x1_ref.shape[1]
    logits = (jnp.dot(x1_ref[...], wt_ref[0:h2],
                      preferred_element_type=jnp.float32) +
              jnp.dot(x2_ref[...], wt_ref[h2:2 * h2],
                      preferred_element_type=jnp.float32))
    s_out_ref[...] = jax.nn.sigmoid(logits)


def _tc_scores(x, wt, chunk, n_chunks):
    tokens, hidden = x.shape
    n_experts = wt.shape[1]
    tb = 1024
    ctokens = tokens // n_chunks
    blk0 = chunk * (ctokens // tb)
    return pl.pallas_call(
        _scores_body,
        grid=(ctokens // tb,),
        in_specs=[
            pl.BlockSpec((tb, hidden // 2), lambda i: (blk0 + i, 0)),
            pl.BlockSpec((tb, hidden // 2), lambda i: (blk0 + i, 1)),
            pl.BlockSpec((hidden, n_experts), lambda i: (0, 0)),
        ],
        out_specs=pl.BlockSpec((tb, n_experts), lambda i: (i, 0)),
        out_shape=jax.ShapeDtypeStruct((ctokens, n_experts), jnp.float32),
    )(x, x, wt)


def _sc_topk(scores, router_bias, *, tokens, n_experts):
    info = plsc.get_sparse_core_info()
    nc, ns, nl = info.num_cores, info.num_subcores, info.num_lanes
    nw = nc * ns
    tpw = tokens // nw
    mesh = plsc.VectorSubcoreMesh(core_axis_name="c", subcore_axis_name="s")

    @functools.partial(
        pl.kernel, mesh=mesh,
        out_type=[
            jax.ShapeDtypeStruct((tokens * TOPK,), jnp.float32),
            jax.ShapeDtypeStruct((tokens * TOPK,), jnp.int32),
        ],
        scratch_types=[
            pltpu.VMEM((tpw * n_experts,), jnp.float32),
            pltpu.VMEM((n_experts,), jnp.float32),
            pltpu.VMEM((tpw * TOPK + nl,), jnp.float32),
            pltpu.VMEM((tpw * TOPK + nl,), jnp.int32),
            pltpu.SemaphoreType.DMA,
        ],
        compiler_params=pltpu.CompilerParams(needs_layout_passes=False),
    )
    def k(scores_hbm, bias_hbm, w_hbm, i_hbm, sc_v, bias_v, wout_v, iout_v,
          sem):
        lane = lax.iota(jnp.int32, nl)
        lo_mask = lane < TOPK
        wid = lax.axis_index("s") * nc + lax.axis_index("c")
        base = wid * tpw
        pltpu.sync_copy(scores_hbm.at[pl.ds(base * n_experts,
                                            tpw * n_experts)], sc_v)
        pltpu.sync_copy(bias_hbm, bias_v)

        bias_vregs = [bias_v[pl.ds(j * nl, nl)] for j in range(4)]

        @plsc.parallel_loop(0, tpw, 1, unroll=4)
        def body(t):
            off = t * n_experts
            ks, vs = [], []
            for j in range(4):
                kj = sc_v[pl.ds(off + j * nl, nl)] + bias_vregs[j]
                vj = lane + j * nl
                sk, sv = plsc.sort_key_val(kj, vj, descending=(j % 2 == 0))
                ks.append(sk)
                vs.append(sv)
            k01 = jnp.where(lo_mask, ks[0], ks[1])
            v01 = jnp.where(lo_mask, vs[0], vs[1])
            k23 = jnp.where(lo_mask, ks[2], ks[3])
            v23 = jnp.where(lo_mask, vs[2], vs[3])
            k01, v01 = plsc.sort_key_val(k01, v01, descending=True)
            k23, v23 = plsc.sort_key_val(k23, v23, descending=False)
            kf = jnp.where(lo_mask, k01, k23)
            vf = jnp.where(lo_mask, v01, v23)
            kf, vf = plsc.sort_key_val(kf, vf, descending=True)
            bsel = plsc.load_gather(bias_v, [vf])
            w = jnp.abs(kf - bsel)
            wm = jnp.where(lo_mask, w, 0.0)
            cs = plsc.cumsum(wm)
            l1 = jnp.maximum(lax.rev(cs, (0,)), 1e-12)
            plsc.store_compressed(wout_v.at[pl.ds(t * TOPK, nl)], wm / l1,
                                  mask=lo_mask)
            plsc.store_compressed(iout_v.at[pl.ds(t * TOPK, nl)], vf,
                                  mask=lo_mask)

        pltpu.sync_copy(wout_v.at[pl.ds(0, tpw * TOPK)],
                        w_hbm.at[pl.ds(base * TOPK, tpw * TOPK)])
        pltpu.sync_copy(iout_v.at[pl.ds(0, tpw * TOPK)],
                        i_hbm.at[pl.ds(base * TOPK, tpw * TOPK)])

    return k(scores.reshape(tokens * n_experts), router_bias)


def kernel(x, W, router_bias):
    tokens, hidden = x.shape
    n_experts = W.shape[0]
    wt = W.T
    n_chunks = 4
    ctokens = tokens // n_chunks
    scores = [_tc_scores(x, wt, c, n_chunks) for c in range(n_chunks)]
    outs = [_sc_topk(s_, router_bias, tokens=ctokens, n_experts=n_experts)
            for s_ in scores]
    ws = [w.reshape(ctokens, TOPK) for w, _ in outs]
    idxs = [i.reshape(ctokens, TOPK) for _, i in outs]
    return (jnp.concatenate(ws, axis=0), jnp.concatenate(idxs, axis=0))


# speedup vs baseline: 1.0463x; 1.0463x over previous
import functools

import jax
import jax.numpy as jnp
from jax import lax
from jax.experimental import pallas as pl
from jax.experimental.pallas import tpu as pltpu
from jax.experimental.pallas import tpu_sc as plsc

TOPK = 8


def _scores_body(x1_ref, x2_ref, wt_ref, s_out_ref):
    h2 = x1_ref.shape[1]
    logits = (jnp.dot(x1_ref[...], wt_ref[0:h2],
                      preferred_element_type=jnp.float32) +
              jnp.dot(x2_ref[...], wt_ref[h2:2 * h2],
                      preferred_element_type=jnp.float32))
    s_out_ref[...] = jax.nn.sigmoid(logits)


def _tc_scores(x, wt, tok0, ctokens):
    tokens, hidden = x.shape
    n_experts = wt.shape[1]
    tb = 1024
    blk0 = tok0 // tb
    return pl.pallas_call(
        _scores_body,
        grid=(ctokens // tb,),
        in_specs=[
            pl.BlockSpec((tb, hidden // 2), lambda i: (blk0 + i, 0)),
            pl.BlockSpec((tb, hidden // 2), lambda i: (blk0 + i, 1)),
            pl.BlockSpec((hidden, n_experts), lambda i: (0, 0)),
        ],
        out_specs=pl.BlockSpec((tb, n_experts), lambda i: (i, 0)),
        out_shape=jax.ShapeDtypeStruct((ctokens, n_experts), jnp.float32),
    )(x, x, wt)


def _sc_topk(scores, router_bias, *, tokens, n_experts):
    info = plsc.get_sparse_core_info()
    nc, ns, nl = info.num_cores, info.num_subcores, info.num_lanes
    nw = nc * ns
    tpw = tokens // nw
    mesh = plsc.VectorSubcoreMesh(core_axis_name="c", subcore_axis_name="s")

    @functools.partial(
        pl.kernel, mesh=mesh,
        out_type=[
            jax.ShapeDtypeStruct((tokens * TOPK,), jnp.float32),
            jax.ShapeDtypeStruct((tokens * TOPK,), jnp.int32),
        ],
        scratch_types=[
            pltpu.VMEM((tpw * n_experts,), jnp.float32),
            pltpu.VMEM((n_experts,), jnp.float32),
            pltpu.VMEM((tpw * TOPK + nl,), jnp.float32),
            pltpu.VMEM((tpw * TOPK + nl,), jnp.int32),
            pltpu.SemaphoreType.DMA,
        ],
        compiler_params=pltpu.CompilerParams(needs_layout_passes=False),
    )
    def k(scores_hbm, bias_hbm, w_hbm, i_hbm, sc_v, bias_v, wout_v, iout_v,
          sem):
        lane = lax.iota(jnp.int32, nl)
        lo_mask = lane < TOPK
        wid = lax.axis_index("s") * nc + lax.axis_index("c")
        base = wid * tpw
        pltpu.sync_copy(scores_hbm.at[pl.ds(base * n_experts,
                                            tpw * n_experts)], sc_v)
        pltpu.sync_copy(bias_hbm, bias_v)

        bias_vregs = [bias_v[pl.ds(j * nl, nl)] for j in range(4)]

        @plsc.parallel_loop(0, tpw, 1, unroll=4)
        def body(t):
            off = t * n_experts
            ks, vs = [], []
            for j in range(4):
                kj = sc_v[pl.ds(off + j * nl, nl)] + bias_vregs[j]
                vj = lane + j * nl
                sk, sv = plsc.sort_key_val(kj, vj, descending=(j % 2 == 0))
                ks.append(sk)
                vs.append(sv)
            k01 = jnp.where(lo_mask, ks[0], ks[1])
            v01 = jnp.where(lo_mask, vs[0], vs[1])
            k23 = jnp.where(lo_mask, ks[2], ks[3])
            v23 = jnp.where(lo_mask, vs[2], vs[3])
            k01, v01 = plsc.sort_key_val(k01, v01, descending=True)
            k23, v23 = plsc.sort_key_val(k23, v23, descending=False)
            kf = jnp.where(lo_mask, k01, k23)
            vf = jnp.where(lo_mask, v01, v23)
            kf, vf = plsc.sort_key_val(kf, vf, descending=True)
            bsel = plsc.load_gather(bias_v, [vf])
            w = jnp.abs(kf - bsel)
            wm = jnp.where(lo_mask, w, 0.0)
            cs = plsc.cumsum(wm)
            l1 = jnp.maximum(lax.rev(cs, (0,)), 1e-12)
            plsc.store_compressed(wout_v.at[pl.ds(t * TOPK, nl)], wm / l1,
                                  mask=lo_mask)
            plsc.store_compressed(iout_v.at[pl.ds(t * TOPK, nl)], vf,
                                  mask=lo_mask)

        pltpu.sync_copy(wout_v.at[pl.ds(0, tpw * TOPK)],
                        w_hbm.at[pl.ds(base * TOPK, tpw * TOPK)])
        pltpu.sync_copy(iout_v.at[pl.ds(0, tpw * TOPK)],
                        i_hbm.at[pl.ds(base * TOPK, tpw * TOPK)])

    return k(scores.reshape(tokens * n_experts), router_bias)


def kernel(x, W, router_bias):
    tokens, hidden = x.shape
    n_experts = W.shape[0]
    wt = W.T
    splits = (tokens // 2, tokens // 4, tokens // 4)
    starts = (0, tokens // 2, tokens * 3 // 4)
    scores = [_tc_scores(x, wt, t0, ct)
              for t0, ct in zip(starts, splits)]
    outs = [_sc_topk(s_, router_bias, tokens=ct, n_experts=n_experts)
            for s_, ct in zip(scores, splits)]
    ws = [w.reshape(ct, TOPK) for (w, _), ct in zip(outs, splits)]
    idxs = [i.reshape(ct, TOPK) for (_, i), ct in zip(outs, splits)]
    return (jnp.concatenate(ws, axis=0), jnp.concatenate(idxs, axis=0))


# speedup vs baseline: 1.0708x; 1.0235x over previous
import functools

import jax
import jax.numpy as jnp
from jax import lax
from jax.experimental import pallas as pl
from jax.experimental.pallas import tpu as pltpu
from jax.experimental.pallas import tpu_sc as plsc

TOPK = 8


def _scores_body(x1_ref, x2_ref, wt_ref, s_out_ref):
    h2 = x1_ref.shape[1]
    logits = (jnp.dot(x1_ref[...], wt_ref[0:h2],
                      preferred_element_type=jnp.float32) +
              jnp.dot(x2_ref[...], wt_ref[h2:2 * h2],
                      preferred_element_type=jnp.float32))
    s_out_ref[...] = jax.nn.sigmoid(logits)


def _tc_scores(x, wt, tok0, ctokens):
    tokens, hidden = x.shape
    n_experts = wt.shape[1]
    tb = 1024
    blk0 = tok0 // tb
    return pl.pallas_call(
        _scores_body,
        grid=(ctokens // tb,),
        in_specs=[
            pl.BlockSpec((tb, hidden // 2), lambda i: (blk0 + i, 0)),
            pl.BlockSpec((tb, hidden // 2), lambda i: (blk0 + i, 1)),
            pl.BlockSpec((hidden, n_experts), lambda i: (0, 0)),
        ],
        out_specs=pl.BlockSpec((tb, n_experts), lambda i: (i, 0)),
        out_shape=jax.ShapeDtypeStruct((ctokens, n_experts), jnp.float32),
    )(x, x, wt)


def _sc_topk(scores, router_bias, *, tokens, n_experts):
    info = plsc.get_sparse_core_info()
    nc, ns, nl = info.num_cores, info.num_subcores, info.num_lanes
    nw = nc * ns
    tpw = tokens // nw
    mesh = plsc.VectorSubcoreMesh(core_axis_name="c", subcore_axis_name="s")

    @functools.partial(
        pl.kernel, mesh=mesh,
        out_type=[
            jax.ShapeDtypeStruct((tokens * TOPK,), jnp.float32),
            jax.ShapeDtypeStruct((tokens * TOPK,), jnp.int32),
        ],
        scratch_types=[
            pltpu.VMEM((tpw * n_experts,), jnp.float32),
            pltpu.VMEM((n_experts,), jnp.float32),
            pltpu.VMEM((tpw * TOPK + nl,), jnp.float32),
            pltpu.VMEM((tpw * TOPK + nl,), jnp.int32),
            pltpu.SemaphoreType.DMA,
        ],
        compiler_params=pltpu.CompilerParams(needs_layout_passes=False),
    )
    def k(scores_hbm, bias_hbm, w_hbm, i_hbm, sc_v, bias_v, wout_v, iout_v,
          sem):
        lane = lax.iota(jnp.int32, nl)
        lo_mask = lane < TOPK
        wid = lax.axis_index("s") * nc + lax.axis_index("c")
        base = wid * tpw
        pltpu.sync_copy(scores_hbm.at[pl.ds(base * n_experts,
                                            tpw * n_experts)], sc_v)
        pltpu.sync_copy(bias_hbm, bias_v)

        bias_vregs = [bias_v[pl.ds(j * nl, nl)] for j in range(4)]

        @plsc.parallel_loop(0, tpw, 1, unroll=4)
        def body(t):
            off = t * n_experts
            ks, vs = [], []
            for j in range(4):
                kj = sc_v[pl.ds(off + j * nl, nl)] + bias_vregs[j]
                vj = lane + j * nl
                sk, sv = plsc.sort_key_val(kj, vj, descending=(j % 2 == 0))
                ks.append(sk)
                vs.append(sv)
            k01 = jnp.where(lo_mask, ks[0], ks[1])
            v01 = jnp.where(lo_mask, vs[0], vs[1])
            k23 = jnp.where(lo_mask, ks[2], ks[3])
            v23 = jnp.where(lo_mask, vs[2], vs[3])
            k01, v01 = plsc.sort_key_val(k01, v01, descending=True)
            k23, v23 = plsc.sort_key_val(k23, v23, descending=False)
            kf = jnp.where(lo_mask, k01, k23)
            vf = jnp.where(lo_mask, v01, v23)
            kf, vf = plsc.sort_key_val(kf, vf, descending=True)
            bsel = plsc.load_gather(bias_v, [vf])
            w = jnp.abs(kf - bsel)
            wm = jnp.where(lo_mask, w, 0.0)
            cs = plsc.cumsum(wm)
            l1 = jnp.maximum(lax.rev(cs, (0,)), 1e-12)
            plsc.store_compressed(wout_v.at[pl.ds(t * TOPK, nl)], wm / l1,
                                  mask=lo_mask)
            plsc.store_compressed(iout_v.at[pl.ds(t * TOPK, nl)], vf,
                                  mask=lo_mask)

        pltpu.sync_copy(wout_v.at[pl.ds(0, tpw * TOPK)],
                        w_hbm.at[pl.ds(base * TOPK, tpw * TOPK)])
        pltpu.sync_copy(iout_v.at[pl.ds(0, tpw * TOPK)],
                        i_hbm.at[pl.ds(base * TOPK, tpw * TOPK)])

    return k(scores.reshape(tokens * n_experts), router_bias)


def kernel(x, W, router_bias):
    tokens, hidden = x.shape
    n_experts = W.shape[0]
    wt = W.T
    splits = (tokens // 2, tokens // 2)
    starts = (0, tokens // 2)
    scores = [_tc_scores(x, wt, t0, ct)
              for t0, ct in zip(starts, splits)]
    outs = [_sc_topk(s_, router_bias, tokens=ct, n_experts=n_experts)
            for s_, ct in zip(scores, splits)]
    ws = [w.reshape(ct, TOPK) for (w, _), ct in zip(outs, splits)]
    idxs = [i.reshape(ct, TOPK) for (_, i), ct in zip(outs, splits)]
    return (jnp.concatenate(ws, axis=0), jnp.concatenate(idxs, axis=0))


# speedup vs baseline: 1.0732x; 1.0022x over previous
import functools

import jax
import jax.numpy as jnp
from jax import lax
from jax.experimental import pallas as pl
from jax.experimental.pallas import tpu as pltpu
from jax.experimental.pallas import tpu_sc as plsc

TOPK = 8


def _scores_body(x1_ref, x2_ref, wt_ref, s_out_ref):
    h2 = x1_ref.shape[1]
    logits = (jnp.dot(x1_ref[...], wt_ref[0:h2],
                      preferred_element_type=jnp.float32) +
              jnp.dot(x2_ref[...], wt_ref[h2:2 * h2],
                      preferred_element_type=jnp.float32))
    s_out_ref[...] = jax.nn.sigmoid(logits)


def _tc_scores(x, wt, tok0, ctokens):
    tokens, hidden = x.shape
    n_experts = wt.shape[1]
    tb = 1024
    blk0 = tok0 // tb
    return pl.pallas_call(
        _scores_body,
        grid=(ctokens // tb,),
        in_specs=[
            pl.BlockSpec((tb, hidden // 2), lambda i: (blk0 + i, 0)),
            pl.BlockSpec((tb, hidden // 2), lambda i: (blk0 + i, 1)),
            pl.BlockSpec((hidden, n_experts), lambda i: (0, 0)),
        ],
        out_specs=pl.BlockSpec((tb, n_experts), lambda i: (i, 0)),
        out_shape=jax.ShapeDtypeStruct((ctokens, n_experts), jnp.float32),
    )(x, x, wt)


def _sc_topk(scores, router_bias, *, tokens, n_experts):
    info = plsc.get_sparse_core_info()
    nc, ns, nl = info.num_cores, info.num_subcores, info.num_lanes
    nw = nc * ns
    tpw = tokens // nw
    mesh = plsc.VectorSubcoreMesh(core_axis_name="c", subcore_axis_name="s")

    @functools.partial(
        pl.kernel, mesh=mesh,
        out_type=[
            jax.ShapeDtypeStruct((tokens * TOPK,), jnp.float32),
            jax.ShapeDtypeStruct((tokens * TOPK,), jnp.int32),
        ],
        scratch_types=[
            pltpu.VMEM((tpw * n_experts,), jnp.float32),
            pltpu.VMEM((n_experts,), jnp.float32),
            pltpu.VMEM((tpw * TOPK + nl,), jnp.float32),
            pltpu.VMEM((tpw * TOPK + nl,), jnp.int32),
            pltpu.SemaphoreType.DMA,
        ],
        compiler_params=pltpu.CompilerParams(needs_layout_passes=False),
    )
    def k(scores_hbm, bias_hbm, w_hbm, i_hbm, sc_v, bias_v, wout_v, iout_v,
          sem):
        lane = lax.iota(jnp.int32, nl)
        lo_mask = lane < TOPK
        wid = lax.axis_index("s") * nc + lax.axis_index("c")
        base = wid * tpw
        pltpu.sync_copy(scores_hbm.at[pl.ds(base * n_experts,
                                            tpw * n_experts)], sc_v)
        pltpu.sync_copy(bias_hbm, bias_v)

        bias_vregs = [bias_v[pl.ds(j * nl, nl)] for j in range(4)]

        @plsc.parallel_loop(0, tpw, 1, unroll=4)
        def body(t):
            off = t * n_experts
            ks, vs = [], []
            for j in range(4):
                kj = sc_v[pl.ds(off + j * nl, nl)] + bias_vregs[j]
                vj = lane + j * nl
                sk, sv = plsc.sort_key_val(kj, vj, descending=(j % 2 == 0))
                ks.append(sk)
                vs.append(sv)
            k01 = jnp.where(lo_mask, ks[0], ks[1])
            v01 = jnp.where(lo_mask, vs[0], vs[1])
            k23 = jnp.where(lo_mask, ks[2], ks[3])
            v23 = jnp.where(lo_mask, vs[2], vs[3])
            k01, v01 = plsc.sort_key_val(k01, v01, descending=True)
            k23, v23 = plsc.sort_key_val(k23, v23, descending=False)
            kf = jnp.where(lo_mask, k01, k23)
            vf = jnp.where(lo_mask, v01, v23)
            kf, vf = plsc.sort_key_val(kf, vf, descending=True)
            bsel = plsc.load_gather(bias_v, [vf])
            w = jnp.abs(kf - bsel)
            wm = jnp.where(lo_mask, w, 0.0)
            cs = plsc.cumsum(wm)
            l1 = jnp.maximum(lax.rev(cs, (0,)), 1e-12)
            plsc.store_compressed(wout_v.at[pl.ds(t * TOPK, nl)], wm / l1,
                                  mask=lo_mask)
            plsc.store_compressed(iout_v.at[pl.ds(t * TOPK, nl)], vf,
                                  mask=lo_mask)

        pltpu.sync_copy(wout_v.at[pl.ds(0, tpw * TOPK)],
                        w_hbm.at[pl.ds(base * TOPK, tpw * TOPK)])
        pltpu.sync_copy(iout_v.at[pl.ds(0, tpw * TOPK)],
                        i_hbm.at[pl.ds(base * TOPK, tpw * TOPK)])

    return k(scores.reshape(tokens * n_experts), router_bias)


def kernel(x, W, router_bias):
    tokens, hidden = x.shape
    n_experts = W.shape[0]
    wt = W.T
    splits = (tokens // 2, tokens // 2)
    starts = (0, tokens // 2)
    s0 = _tc_scores(x, wt, starts[0], splits[0])
    o0 = _sc_topk(s0, router_bias, tokens=splits[0], n_experts=n_experts)
    s1 = _tc_scores(x, wt, starts[1], splits[1])
    o1 = _sc_topk(s1, router_bias, tokens=splits[1], n_experts=n_experts)
    outs = [o0, o1]
    ws = [w.reshape(ct, TOPK) for (w, _), ct in zip(outs, splits)]
    idxs = [i.reshape(ct, TOPK) for (_, i), ct in zip(outs, splits)]
    return (jnp.concatenate(ws, axis=0), jnp.concatenate(idxs, axis=0))
